# baseline (speedup 1.0000x reference)
import jax
import jax.numpy as jnp
from jax import lax
from jax.experimental import pallas as pl
from jax.experimental.pallas import tpu as pltpu
from jax.experimental.pallas import tpu_sc as plsc

E = 8
TOP_K = 2
H = 1024
D_FF = 2048
T = 2048

BLK = 256
CAP = TOP_K * T + E * BLK
NBLK = CAP // BLK
NW = 32
PAIRS_PER_W = (TOP_K * T) // NW
TOK_PER_W = T // NW


def _cumsum_roll(x, axis, n):
    idx = lax.broadcasted_iota(jnp.int32, x.shape, axis)
    s = 1
    while s < n:
        x = x + jnp.where(idx >= s, pltpu.roll(x, s, axis=axis), 0)
        s *= 2
    return x


def _router_body(hs_ref, rw_ref, scores_ref, dest_ref, sx_ref, blk_ref):
    hs = hs_ref[...]
    rw = rw_ref[...]
    logits = lax.dot_general(rw, hs, (((1,), (1,)), ((), ())),
                             preferred_element_type=jnp.float32)
    e_iota = lax.broadcasted_iota(jnp.int32, (E, T), 0)
    m1 = jnp.max(logits, axis=0, keepdims=True)
    i1 = jnp.min(jnp.where(logits == m1, e_iota, E), axis=0, keepdims=True)
    masked = jnp.where(e_iota == i1, -jnp.inf, logits)
    m2 = jnp.max(masked, axis=0, keepdims=True)
    i2 = jnp.min(jnp.where(masked == m2, e_iota, E), axis=0, keepdims=True)

    sel1 = (e_iota == i1)
    sel2 = (e_iota == i2)
    sig = jax.nn.sigmoid(logits)
    gsel1 = jnp.where(sel1, sig, 0.0)
    gsel2 = jnp.where(sel2, sig, 0.0)
    scores_ref[...] = gsel1 + gsel2

    ones_e = jnp.ones((E, 1), jnp.float32)
    g1c = lax.dot_general(gsel1, ones_e, (((0,), (0,)), ((), ())),
                          preferred_element_type=jnp.float32)
    g2c = lax.dot_general(gsel2, ones_e, (((0,), (0,)), ((), ())),
                          preferred_element_type=jnp.float32)
    def _pack(v):
        vb = v.astype(jnp.bfloat16)
        lo = pltpu.bitcast(vb[:, :H // 2], jnp.uint16).astype(jnp.uint32)
        hi = pltpu.bitcast(vb[:, H // 2:], jnp.uint16).astype(jnp.uint32)
        return pltpu.bitcast(lo | (hi << 16), jnp.int32)

    sx_ref[0:T, :] = _pack(hs * g1c)
    sx_ref[T:2 * T, :] = _pack(hs * g2c)

    s1 = sel1.astype(jnp.int32)
    s2 = sel2.astype(jnp.int32)
    c1 = _cumsum_roll(s1, 1, T)
    c2 = _cumsum_roll(s2, 1, T)
    cnt1 = c1[:, T - 1:T]
    cnt = cnt1 + c2[:, T - 1:T]
    padded = ((cnt + (BLK - 1)) // BLK) * BLK
    cum_pad = _cumsum_roll(padded, 0, E)
    pad_off = cum_pad - padded

    rank1 = c1 - s1
    rank2 = cnt1 + c2 - s2
    dest_ref[0:1, :] = jnp.sum(s1 * (pad_off + rank1), axis=0, keepdims=True)
    dest_ref[1:2, :] = jnp.sum(s2 * (pad_off + rank2), axis=0, keepdims=True)

    bstart = lax.broadcasted_iota(jnp.int32, (E, NBLK), 1) * BLK
    be = jnp.sum((cum_pad <= bstart).astype(jnp.int32), axis=0, keepdims=True)
    blk_ref[...] = jnp.minimum(be, E - 1)


def _router(hs, router_w):
    return pl.pallas_call(
        _router_body,
        out_shape=(
            jax.ShapeDtypeStruct((E, T), jnp.float32),
            jax.ShapeDtypeStruct((TOP_K, T), jnp.int32),
            jax.ShapeDtypeStruct((TOP_K * T, H // 2), jnp.int32),
            jax.ShapeDtypeStruct((1, NBLK), jnp.int32),
        ),
    )(hs, router_w)


_DCHUNK = 64


def _dispatch_body(sx_hbm, dest_hbm, gx_hbm,
                   x0_v, x1_v, i0_v, i1_v, lsem, ssem):
    wid = lax.axis_index("s") * 2 + lax.axis_index("c")
    k_half = wid // 16
    toff = (wid % 16) * PAIRS_PER_W
    t0 = toff
    t1 = toff + _DCHUNK
    l0a = pltpu.async_copy(sx_hbm.at[pl.ds(k_half * T + t0, _DCHUNK), :], x0_v, lsem)
    l0b = pltpu.async_copy(dest_hbm.at[k_half, pl.ds(t0, _DCHUNK)], i0_v, lsem)
    l1a = pltpu.async_copy(sx_hbm.at[pl.ds(k_half * T + t1, _DCHUNK), :], x1_v, lsem)
    l1b = pltpu.async_copy(dest_hbm.at[k_half, pl.ds(t1, _DCHUNK)], i1_v, lsem)
    l0a.wait()
    l0b.wait()
    s0 = pltpu.async_copy(x0_v, gx_hbm.at[i0_v], ssem)
    l1a.wait()
    l1b.wait()
    s1 = pltpu.async_copy(x1_v, gx_hbm.at[i1_v], ssem)
    s0.wait()
    s1.wait()


def _dispatch(sx, dest):
    mesh = plsc.VectorSubcoreMesh(core_axis_name="c", subcore_axis_name="s")
    return pl.kernel(
        _dispatch_body,
        out_type=jax.ShapeDtypeStruct((CAP, H // 2), jnp.int32),
        mesh=mesh,
        scratch_types=[
            pltpu.VMEM((_DCHUNK, H // 2), jnp.int32),
            pltpu.VMEM((_DCHUNK, H // 2), jnp.int32),
            pltpu.VMEM((_DCHUNK,), jnp.int32),
            pltpu.VMEM((_DCHUNK,), jnp.int32),
            pltpu.SemaphoreType.DMA,
            pltpu.SemaphoreType.DMA,
        ],
    )(sx, dest)


def _mlp_block(x, gw, uw, dw):
    xb = x.astype(jnp.bfloat16)
    a = lax.dot_general(xb, gw.astype(jnp.bfloat16), (((1,), (1,)), ((), ())),
                        preferred_element_type=jnp.float32)
    b = lax.dot_general(xb, uw.astype(jnp.bfloat16), (((1,), (1,)), ((), ())),
                        preferred_element_type=jnp.float32)
    h = a * jax.nn.sigmoid(a) * b
    return lax.dot_general(h.astype(jnp.bfloat16), dw.astype(jnp.bfloat16),
                           (((1,), (1,)), ((), ())),
                           preferred_element_type=jnp.float32)


def _shared_body(x_ref, gw_ref, uw_ref, dw_ref, y_ref):
    y_ref[...] = _mlp_block(x_ref[...], gw_ref[...], uw_ref[...], dw_ref[...])


_BLK_S = 256


def _shared_mlp(hs, sgw, suw, sdw):
    return pl.pallas_call(
        _shared_body,
        grid=(T // _BLK_S,),
        in_specs=[
            pl.BlockSpec((_BLK_S, H), lambda i: (i, 0)),
            pl.BlockSpec((D_FF, H), lambda i: (0, 0)),
            pl.BlockSpec((D_FF, H), lambda i: (0, 0)),
            pl.BlockSpec((H, D_FF), lambda i: (0, 0)),
        ],
        out_specs=pl.BlockSpec((_BLK_S, H), lambda i: (i, 0)),
        out_shape=jax.ShapeDtypeStruct((T, H), jnp.float32),
    )(hs, sgw, suw, sdw)


def _group_body(be_ref, x_ref, gw_ref, uw_ref, dw_ref, y_ref):
    w = pltpu.bitcast(x_ref[...], jnp.uint32)
    lo = pltpu.bitcast(w.astype(jnp.uint16), jnp.bfloat16)
    hi = pltpu.bitcast((w >> 16).astype(jnp.uint16), jnp.bfloat16)
    x = jnp.concatenate([lo, hi], axis=1)
    y_ref[...] = _mlp_block(x, gw_ref[0], uw_ref[0], dw_ref[0])


def _grouped_mlp(blk_e, gx, gate_w, up_w, down_w):
    grid_spec = pltpu.PrefetchScalarGridSpec(
        num_scalar_prefetch=1,
        grid=(NBLK,),
        in_specs=[
            pl.BlockSpec((BLK, H // 2), lambda i, be: (i, 0)),
            pl.BlockSpec((1, D_FF, H), lambda i, be: (be[0, i], 0, 0)),
            pl.BlockSpec((1, D_FF, H), lambda i, be: (be[0, i], 0, 0)),
            pl.BlockSpec((1, H, D_FF), lambda i, be: (be[0, i], 0, 0)),
        ],
        out_specs=pl.BlockSpec((BLK, H), lambda i, be: (i, 0)),
    )
    return pl.pallas_call(
        _group_body,
        grid_spec=grid_spec,
        out_shape=jax.ShapeDtypeStruct((CAP, H), jnp.float32),
    )(blk_e, gx, gate_w, up_w, down_w)


_CCHUNK = 8
_CNBUF = 4
_CN = TOK_PER_W // _CCHUNK


def _combine_body(y_hbm, ysh_hbm, dest_hbm, out_hbm,
                  a_v, b_v, c_v, i1_v, i2_v, gsem, osem):
    wid = lax.axis_index("s") * 2 + lax.axis_index("c")
    base = wid * TOK_PER_W

    def _issue(c, buf):
        t0 = base + c * _CCHUNK
        pltpu.sync_copy(dest_hbm.at[0, pl.ds(t0, _CCHUNK)], i1_v.at[buf])
        pltpu.sync_copy(dest_hbm.at[1, pl.ds(t0, _CCHUNK)], i2_v.at[buf])
        return (pltpu.async_copy(y_hbm.at[i1_v.at[buf]], a_v.at[buf], gsem),
                pltpu.async_copy(y_hbm.at[i2_v.at[buf]], b_v.at[buf], gsem),
                pltpu.async_copy(ysh_hbm.at[pl.ds(t0, _CCHUNK), :],
                                 c_v.at[buf], gsem))

    pend = [None] * _CN
    outcp = [None] * _CN
    for c in range(_CNBUF - 1):
        pend[c] = _issue(c, c % _CNBUF)
    for c in range(_CN):
        if c + _CNBUF - 1 < _CN:
            if c - 1 >= 0:
                outcp[c - 1].wait()
            pend[c + _CNBUF - 1] = _issue(c + _CNBUF - 1,
                                          (c + _CNBUF - 1) % _CNBUF)
        for cp in pend[c]:
            cp.wait()
        buf = c % _CNBUF

        def _row(j, _):
            for cb in range(H // 16):
                sl = pl.ds(cb * 16, 16)
                a_v[buf, j, sl] = a_v[buf, j, sl] + b_v[buf, j, sl] + c_v[buf, j, sl]
            return 0

        lax.fori_loop(0, _CCHUNK, _row, 0)
        outcp[c] = pltpu.async_copy(
            a_v.at[buf], out_hbm.at[pl.ds(base + c * _CCHUNK, _CCHUNK), :], osem)
    for c in range(max(0, _CN - _CNBUF), _CN):
        outcp[c].wait()


def _combine(y, ysh, dest):
    mesh = plsc.VectorSubcoreMesh(core_axis_name="c", subcore_axis_name="s")
    return pl.kernel(
        _combine_body,
        out_type=jax.ShapeDtypeStruct((T, H), jnp.float32),
        mesh=mesh,
        scratch_types=[
            pltpu.VMEM((_CNBUF, _CCHUNK, H), jnp.float32),
            pltpu.VMEM((_CNBUF, _CCHUNK, H), jnp.float32),
            pltpu.VMEM((_CNBUF, _CCHUNK, H), jnp.float32),
            pltpu.VMEM((_CNBUF, _CCHUNK), jnp.int32),
            pltpu.VMEM((_CNBUF, _CCHUNK), jnp.int32),
            pltpu.SemaphoreType.DMA,
            pltpu.SemaphoreType.DMA,
        ],
    )(y, ysh, dest)


def kernel(hidden_states, router_w, gate_w, up_w, down_w,
           shared_gate_w, shared_up_w, shared_down_w):
    b, s, hd = hidden_states.shape
    hs = hidden_states.reshape(-1, hd)
    router_scores, dest, sx, blk_e = _router(hs, router_w)
    ysh = _shared_mlp(hs, shared_gate_w, shared_up_w, shared_down_w)
    gx = _dispatch(sx, dest)
    y = _grouped_mlp(blk_e, gx, gate_w, up_w, down_w)
    out = _combine(y, ysh, dest)
    return out, router_scores


# speedup vs baseline: 1.0570x; 1.0570x over previous
import jax
import jax.numpy as jnp
from jax import lax
from jax.experimental import pallas as pl
from jax.experimental.pallas import tpu as pltpu
from jax.experimental.pallas import tpu_sc as plsc

E = 8
TOP_K = 2
H = 1024
D_FF = 2048
T = 2048

BLK = 256
CAP = TOP_K * T + E * BLK
NBLK = CAP // BLK
NW = 32
PAIRS_PER_W = (TOP_K * T) // NW
TOK_PER_W = T // NW


def _cumsum_roll(x, axis, n):
    idx = lax.broadcasted_iota(jnp.int32, x.shape, axis)
    s = 1
    while s < n:
        x = x + jnp.where(idx >= s, pltpu.roll(x, s, axis=axis), 0)
        s *= 2
    return x


def _router_body(hs_ref, rw_ref, scores_ref, dest_ref, sx_ref, blk_ref):
    hs = hs_ref[...]
    rw = rw_ref[...]
    logits = lax.dot_general(rw, hs, (((1,), (1,)), ((), ())),
                             preferred_element_type=jnp.float32)
    e_iota = lax.broadcasted_iota(jnp.int32, (E, T), 0)
    m1 = jnp.max(logits, axis=0, keepdims=True)
    i1 = jnp.min(jnp.where(logits == m1, e_iota, E), axis=0, keepdims=True)
    masked = jnp.where(e_iota == i1, -jnp.inf, logits)
    m2 = jnp.max(masked, axis=0, keepdims=True)
    i2 = jnp.min(jnp.where(masked == m2, e_iota, E), axis=0, keepdims=True)

    sel1 = (e_iota == i1)
    sel2 = (e_iota == i2)
    sig = jax.nn.sigmoid(logits)
    gsel1 = jnp.where(sel1, sig, 0.0)
    gsel2 = jnp.where(sel2, sig, 0.0)
    scores_ref[...] = gsel1 + gsel2

    ones_e = jnp.ones((E, 1), jnp.float32)
    g1c = lax.dot_general(gsel1, ones_e, (((0,), (0,)), ((), ())),
                          preferred_element_type=jnp.float32)
    g2c = lax.dot_general(gsel2, ones_e, (((0,), (0,)), ((), ())),
                          preferred_element_type=jnp.float32)
    def _pack(v):
        vb = v.astype(jnp.bfloat16)
        lo = pltpu.bitcast(vb[:, :H // 2], jnp.uint16).astype(jnp.uint32)
        hi = pltpu.bitcast(vb[:, H // 2:], jnp.uint16).astype(jnp.uint32)
        return pltpu.bitcast(lo | (hi << 16), jnp.int32)

    sx_ref[0:T, :] = _pack(hs * g1c)
    sx_ref[T:2 * T, :] = _pack(hs * g2c)

    s1 = sel1.astype(jnp.int32)
    s2 = sel2.astype(jnp.int32)
    c1 = _cumsum_roll(s1, 1, T)
    c2 = _cumsum_roll(s2, 1, T)
    cnt1 = c1[:, T - 1:T]
    cnt = cnt1 + c2[:, T - 1:T]
    padded = ((cnt + (BLK - 1)) // BLK) * BLK
    cum_pad = _cumsum_roll(padded, 0, E)
    pad_off = cum_pad - padded

    rank1 = c1 - s1
    rank2 = cnt1 + c2 - s2
    dest_ref[0:1, :] = jnp.sum(s1 * (pad_off + rank1), axis=0, keepdims=True)
    dest_ref[1:2, :] = jnp.sum(s2 * (pad_off + rank2), axis=0, keepdims=True)

    bstart = lax.broadcasted_iota(jnp.int32, (E, NBLK), 1) * BLK
    be = jnp.sum((cum_pad <= bstart).astype(jnp.int32), axis=0, keepdims=True)
    blk_ref[...] = jnp.minimum(be, E - 1)


def _router(hs, router_w):
    return pl.pallas_call(
        _router_body,
        out_shape=(
            jax.ShapeDtypeStruct((E, T), jnp.float32),
            jax.ShapeDtypeStruct((TOP_K, T), jnp.int32),
            jax.ShapeDtypeStruct((TOP_K * T, H // 2), jnp.int32),
            jax.ShapeDtypeStruct((1, NBLK), jnp.int32),
        ),
    )(hs, router_w)


_DCHUNK = 64


def _dispatch_body(sx_hbm, dest_hbm, gx_hbm,
                   x0_v, x1_v, i0_v, i1_v, lsem, ssem):
    wid = lax.axis_index("s") * 2 + lax.axis_index("c")
    k_half = wid // 16
    toff = (wid % 16) * PAIRS_PER_W
    t0 = toff
    t1 = toff + _DCHUNK
    l0a = pltpu.async_copy(sx_hbm.at[pl.ds(k_half * T + t0, _DCHUNK), :], x0_v, lsem)
    l0b = pltpu.async_copy(dest_hbm.at[k_half, pl.ds(t0, _DCHUNK)], i0_v, lsem)
    l1a = pltpu.async_copy(sx_hbm.at[pl.ds(k_half * T + t1, _DCHUNK), :], x1_v, lsem)
    l1b = pltpu.async_copy(dest_hbm.at[k_half, pl.ds(t1, _DCHUNK)], i1_v, lsem)
    l0a.wait()
    l0b.wait()
    s0 = pltpu.async_copy(x0_v, gx_hbm.at[i0_v], ssem)
    l1a.wait()
    l1b.wait()
    s1 = pltpu.async_copy(x1_v, gx_hbm.at[i1_v], ssem)
    s0.wait()
    s1.wait()


def _dispatch(sx, dest):
    mesh = plsc.VectorSubcoreMesh(core_axis_name="c", subcore_axis_name="s")
    return pl.kernel(
        _dispatch_body,
        out_type=jax.ShapeDtypeStruct((CAP, H // 2), jnp.int32),
        mesh=mesh,
        scratch_types=[
            pltpu.VMEM((_DCHUNK, H // 2), jnp.int32),
            pltpu.VMEM((_DCHUNK, H // 2), jnp.int32),
            pltpu.VMEM((_DCHUNK,), jnp.int32),
            pltpu.VMEM((_DCHUNK,), jnp.int32),
            pltpu.SemaphoreType.DMA,
            pltpu.SemaphoreType.DMA,
        ],
    )(sx, dest)


def _mlp_block(x, gw, uw, dw):
    xb = x.astype(jnp.bfloat16)
    a = lax.dot_general(xb, gw.astype(jnp.bfloat16), (((1,), (1,)), ((), ())),
                        preferred_element_type=jnp.float32)
    b = lax.dot_general(xb, uw.astype(jnp.bfloat16), (((1,), (1,)), ((), ())),
                        preferred_element_type=jnp.float32)
    h = a * jax.nn.sigmoid(a) * b
    return lax.dot_general(h.astype(jnp.bfloat16), dw.astype(jnp.bfloat16),
                           (((1,), (1,)), ((), ())),
                           preferred_element_type=jnp.float32)


def _shared_body(x_ref, gw_ref, uw_ref, dw_ref, y_ref):
    y_ref[...] = _mlp_block(x_ref[...], gw_ref[...], uw_ref[...], dw_ref[...])


_BLK_S = 256


def _shared_mlp(hs, sgw, suw, sdw):
    return pl.pallas_call(
        _shared_body,
        grid=(T // _BLK_S,),
        in_specs=[
            pl.BlockSpec((_BLK_S, H), lambda i: (i, 0)),
            pl.BlockSpec((D_FF, H), lambda i: (0, 0)),
            pl.BlockSpec((D_FF, H), lambda i: (0, 0)),
            pl.BlockSpec((H, D_FF), lambda i: (0, 0)),
        ],
        out_specs=pl.BlockSpec((_BLK_S, H), lambda i: (i, 0)),
        out_shape=jax.ShapeDtypeStruct((T, H), jnp.float32),
    )(hs, sgw, suw, sdw)


def _group_body(be_ref, x_ref, gw_ref, uw_ref, dw_ref, y_ref):
    w = pltpu.bitcast(x_ref[...], jnp.uint32)
    lo = pltpu.bitcast(w.astype(jnp.uint16), jnp.bfloat16)
    hi = pltpu.bitcast((w >> 16).astype(jnp.uint16), jnp.bfloat16)
    x = jnp.concatenate([lo, hi], axis=1)
    y_ref[...] = _mlp_block(x, gw_ref[0], uw_ref[0], dw_ref[0])


def _grouped_mlp(blk_e, gx, gate_w, up_w, down_w):
    grid_spec = pltpu.PrefetchScalarGridSpec(
        num_scalar_prefetch=1,
        grid=(NBLK,),
        in_specs=[
            pl.BlockSpec((BLK, H // 2), lambda i, be: (i, 0)),
            pl.BlockSpec((1, D_FF, H), lambda i, be: (be[0, i], 0, 0)),
            pl.BlockSpec((1, D_FF, H), lambda i, be: (be[0, i], 0, 0)),
            pl.BlockSpec((1, H, D_FF), lambda i, be: (be[0, i], 0, 0)),
        ],
        out_specs=pl.BlockSpec((BLK, H), lambda i, be: (i, 0)),
    )
    return pl.pallas_call(
        _group_body,
        grid_spec=grid_spec,
        out_shape=jax.ShapeDtypeStruct((CAP, H), jnp.float32),
    )(blk_e, gx, gate_w, up_w, down_w)


_CCHUNK = 16
_CNBUF = 2
_CN = TOK_PER_W // _CCHUNK


def _combine_body(y_hbm, ysh_hbm, dest_hbm, out_hbm,
                  a_v, b_v, c_v, i1_v, i2_v, gsem, osem):
    wid = lax.axis_index("s") * 2 + lax.axis_index("c")
    base = wid * TOK_PER_W

    def _issue(c, buf):
        t0 = base + c * _CCHUNK
        pltpu.sync_copy(dest_hbm.at[0, pl.ds(t0, _CCHUNK)], i1_v.at[buf])
        pltpu.sync_copy(dest_hbm.at[1, pl.ds(t0, _CCHUNK)], i2_v.at[buf])
        return (pltpu.async_copy(y_hbm.at[i1_v.at[buf]], a_v.at[buf], gsem),
                pltpu.async_copy(y_hbm.at[i2_v.at[buf]], b_v.at[buf], gsem),
                pltpu.async_copy(ysh_hbm.at[pl.ds(t0, _CCHUNK), :],
                                 c_v.at[buf], gsem))

    pend = [None] * _CN
    outcp = [None] * _CN
    for c in range(_CNBUF - 1):
        pend[c] = _issue(c, c % _CNBUF)
    for c in range(_CN):
        if c + _CNBUF - 1 < _CN:
            if c - 1 >= 0:
                outcp[c - 1].wait()
            pend[c + _CNBUF - 1] = _issue(c + _CNBUF - 1,
                                          (c + _CNBUF - 1) % _CNBUF)
        for cp in pend[c]:
            cp.wait()
        buf = c % _CNBUF

        def _row(j, _):
            for cb in range(H // 16):
                sl = pl.ds(cb * 16, 16)
                a_v[buf, j, sl] = a_v[buf, j, sl] + b_v[buf, j, sl] + c_v[buf, j, sl]
            return 0

        lax.fori_loop(0, _CCHUNK, _row, 0)
        outcp[c] = pltpu.async_copy(
            a_v.at[buf], out_hbm.at[pl.ds(base + c * _CCHUNK, _CCHUNK), :], osem)
    for c in range(max(0, _CN - _CNBUF), _CN):
        outcp[c].wait()


def _combine(y, ysh, dest):
    mesh = plsc.VectorSubcoreMesh(core_axis_name="c", subcore_axis_name="s")
    return pl.kernel(
        _combine_body,
        out_type=jax.ShapeDtypeStruct((T, H), jnp.float32),
        mesh=mesh,
        scratch_types=[
            pltpu.VMEM((_CNBUF, _CCHUNK, H), jnp.float32),
            pltpu.VMEM((_CNBUF, _CCHUNK, H), jnp.float32),
            pltpu.VMEM((_CNBUF, _CCHUNK, H), jnp.float32),
            pltpu.VMEM((_CNBUF, _CCHUNK), jnp.int32),
            pltpu.VMEM((_CNBUF, _CCHUNK), jnp.int32),
            pltpu.SemaphoreType.DMA,
            pltpu.SemaphoreType.DMA,
        ],
    )(y, ysh, dest)


def kernel(hidden_states, router_w, gate_w, up_w, down_w,
           shared_gate_w, shared_up_w, shared_down_w):
    b, s, hd = hidden_states.shape
    hs = hidden_states.reshape(-1, hd)
    router_scores, dest, sx, blk_e = _router(hs, router_w)
    ysh = _shared_mlp(hs, shared_gate_w, shared_up_w, shared_down_w)
    gx = _dispatch(sx, dest)
    y = _grouped_mlp(blk_e, gx, gate_w, up_w, down_w)
    out = _combine(y, ysh, dest)
    return out, router_scores


# speedup vs baseline: 1.0611x; 1.0039x over previous
import jax
import jax.numpy as jnp
from jax import lax
from jax.experimental import pallas as pl
from jax.experimental.pallas import tpu as pltpu
from jax.experimental.pallas import tpu_sc as plsc

E = 8
TOP_K = 2
H = 1024
D_FF = 2048
T = 2048

BLK = 256
CAP = TOP_K * T + E * BLK
NBLK = CAP // BLK
NW = 32
PAIRS_PER_W = (TOP_K * T) // NW
TOK_PER_W = T // NW


def _cumsum_roll(x, axis, n):
    idx = lax.broadcasted_iota(jnp.int32, x.shape, axis)
    s = 1
    while s < n:
        x = x + jnp.where(idx >= s, pltpu.roll(x, s, axis=axis), 0)
        s *= 2
    return x


def _router_body(hs_ref, rw_ref, scores_ref, dest_ref, sx_ref, blk_ref):
    k = pl.program_id(0)
    hs = hs_ref[...]
    rw = rw_ref[...]
    logits = lax.dot_general(rw, hs, (((1,), (1,)), ((), ())),
                             preferred_element_type=jnp.float32)
    e_iota = lax.broadcasted_iota(jnp.int32, (E, T), 0)
    m1 = jnp.max(logits, axis=0, keepdims=True)
    i1 = jnp.min(jnp.where(logits == m1, e_iota, E), axis=0, keepdims=True)
    masked = jnp.where(e_iota == i1, -jnp.inf, logits)
    m2 = jnp.max(masked, axis=0, keepdims=True)
    i2 = jnp.min(jnp.where(masked == m2, e_iota, E), axis=0, keepdims=True)

    sel1 = (e_iota == i1)
    sel2 = (e_iota == i2)
    sig = jax.nn.sigmoid(logits)
    gsel1 = jnp.where(sel1, sig, 0.0)
    gsel2 = jnp.where(sel2, sig, 0.0)

    ones_e = jnp.ones((E, 1), jnp.float32)
    gsel = jnp.where(k == 0, gsel1, gsel2)
    gc = lax.dot_general(gsel, ones_e, (((0,), (0,)), ((), ())),
                         preferred_element_type=jnp.float32)

    def _pack(v):
        vb = v.astype(jnp.bfloat16)
        lo = pltpu.bitcast(vb[:, :H // 2], jnp.uint16).astype(jnp.uint32)
        hi = pltpu.bitcast(vb[:, H // 2:], jnp.uint16).astype(jnp.uint32)
        return pltpu.bitcast(lo | (hi << 16), jnp.int32)

    sx_ref[...] = _pack(hs * gc)

    @pl.when(k == TOP_K - 1)
    def _metadata():
        scores_ref[...] = gsel1 + gsel2
        s1 = sel1.astype(jnp.int32)
        s2 = sel2.astype(jnp.int32)
        c1 = _cumsum_roll(s1, 1, T)
        c2 = _cumsum_roll(s2, 1, T)
        cnt1 = c1[:, T - 1:T]
        cnt = cnt1 + c2[:, T - 1:T]
        padded = ((cnt + (BLK - 1)) // BLK) * BLK
        cum_pad = _cumsum_roll(padded, 0, E)
        pad_off = cum_pad - padded

        rank1 = c1 - s1
        rank2 = cnt1 + c2 - s2
        dest_ref[0:1, :] = jnp.sum(s1 * (pad_off + rank1), axis=0, keepdims=True)
        dest_ref[1:2, :] = jnp.sum(s2 * (pad_off + rank2), axis=0, keepdims=True)

        bstart = lax.broadcasted_iota(jnp.int32, (E, NBLK), 1) * BLK
        be = jnp.sum((cum_pad <= bstart).astype(jnp.int32), axis=0, keepdims=True)
        blk_ref[...] = jnp.minimum(be, E - 1)


def _router(hs, router_w):
    return pl.pallas_call(
        _router_body,
        grid=(TOP_K,),
        in_specs=[
            pl.BlockSpec((T, H), lambda k: (0, 0)),
            pl.BlockSpec((E, H), lambda k: (0, 0)),
        ],
        out_specs=(
            pl.BlockSpec((E, T), lambda k: (0, 0)),
            pl.BlockSpec((TOP_K, T), lambda k: (0, 0)),
            pl.BlockSpec((T, H // 2), lambda k: (k, 0)),
            pl.BlockSpec((1, NBLK), lambda k: (0, 0)),
        ),
        out_shape=(
            jax.ShapeDtypeStruct((E, T), jnp.float32),
            jax.ShapeDtypeStruct((TOP_K, T), jnp.int32),
            jax.ShapeDtypeStruct((TOP_K * T, H // 2), jnp.int32),
            jax.ShapeDtypeStruct((1, NBLK), jnp.int32),
        ),
    )(hs, router_w)


_DCHUNK = 64


def _dispatch_body(sx_hbm, dest_hbm, gx_hbm,
                   x0_v, x1_v, i0_v, i1_v, lsem, ssem):
    wid = lax.axis_index("s") * 2 + lax.axis_index("c")
    k_half = wid // 16
    toff = (wid % 16) * PAIRS_PER_W
    t0 = toff
    t1 = toff + _DCHUNK
    l0a = pltpu.async_copy(sx_hbm.at[pl.ds(k_half * T + t0, _DCHUNK), :], x0_v, lsem)
    l0b = pltpu.async_copy(dest_hbm.at[k_half, pl.ds(t0, _DCHUNK)], i0_v, lsem)
    l1a = pltpu.async_copy(sx_hbm.at[pl.ds(k_half * T + t1, _DCHUNK), :], x1_v, lsem)
    l1b = pltpu.async_copy(dest_hbm.at[k_half, pl.ds(t1, _DCHUNK)], i1_v, lsem)
    l0a.wait()
    l0b.wait()
    s0 = pltpu.async_copy(x0_v, gx_hbm.at[i0_v], ssem)
    l1a.wait()
    l1b.wait()
    s1 = pltpu.async_copy(x1_v, gx_hbm.at[i1_v], ssem)
    s0.wait()
    s1.wait()


def _dispatch(sx, dest):
    mesh = plsc.VectorSubcoreMesh(core_axis_name="c", subcore_axis_name="s")
    return pl.kernel(
        _dispatch_body,
        out_type=jax.ShapeDtypeStruct((CAP, H // 2), jnp.int32),
        mesh=mesh,
        scratch_types=[
            pltpu.VMEM((_DCHUNK, H // 2), jnp.int32),
            pltpu.VMEM((_DCHUNK, H // 2), jnp.int32),
            pltpu.VMEM((_DCHUNK,), jnp.int32),
            pltpu.VMEM((_DCHUNK,), jnp.int32),
            pltpu.SemaphoreType.DMA,
            pltpu.SemaphoreType.DMA,
        ],
    )(sx, dest)


def _mlp_block(x, gw, uw, dw):
    xb = x.astype(jnp.bfloat16)
    a = lax.dot_general(xb, gw.astype(jnp.bfloat16), (((1,), (1,)), ((), ())),
                        preferred_element_type=jnp.float32)
    b = lax.dot_general(xb, uw.astype(jnp.bfloat16), (((1,), (1,)), ((), ())),
                        preferred_element_type=jnp.float32)
    h = a * jax.nn.sigmoid(a) * b
    return lax.dot_general(h.astype(jnp.bfloat16), dw.astype(jnp.bfloat16),
                           (((1,), (1,)), ((), ())),
                           preferred_element_type=jnp.float32)


def _shared_body(x_ref, gw_ref, uw_ref, dw_ref, y_ref):
    y_ref[...] = _mlp_block(x_ref[...], gw_ref[...], uw_ref[...], dw_ref[...])


_BLK_S = 256


def _shared_mlp(hs, sgw, suw, sdw):
    return pl.pallas_call(
        _shared_body,
        grid=(T // _BLK_S,),
        in_specs=[
            pl.BlockSpec((_BLK_S, H), lambda i: (i, 0)),
            pl.BlockSpec((D_FF, H), lambda i: (0, 0)),
            pl.BlockSpec((D_FF, H), lambda i: (0, 0)),
            pl.BlockSpec((H, D_FF), lambda i: (0, 0)),
        ],
        out_specs=pl.BlockSpec((_BLK_S, H), lambda i: (i, 0)),
        out_shape=jax.ShapeDtypeStruct((T, H), jnp.float32),
    )(hs, sgw, suw, sdw)


def _group_body(be_ref, x_ref, gw_ref, uw_ref, dw_ref, y_ref):
    w = pltpu.bitcast(x_ref[...], jnp.uint32)
    lo = pltpu.bitcast(w.astype(jnp.uint16), jnp.bfloat16)
    hi = pltpu.bitcast((w >> 16).astype(jnp.uint16), jnp.bfloat16)
    x = jnp.concatenate([lo, hi], axis=1)
    y_ref[...] = _mlp_block(x, gw_ref[0], uw_ref[0], dw_ref[0])


def _grouped_mlp(blk_e, gx, gate_w, up_w, down_w):
    grid_spec = pltpu.PrefetchScalarGridSpec(
        num_scalar_prefetch=1,
        grid=(NBLK,),
        in_specs=[
            pl.BlockSpec((BLK, H // 2), lambda i, be: (i, 0)),
            pl.BlockSpec((1, D_FF, H), lambda i, be: (be[0, i], 0, 0)),
            pl.BlockSpec((1, D_FF, H), lambda i, be: (be[0, i], 0, 0)),
            pl.BlockSpec((1, H, D_FF), lambda i, be: (be[0, i], 0, 0)),
        ],
        out_specs=pl.BlockSpec((BLK, H), lambda i, be: (i, 0)),
    )
    return pl.pallas_call(
        _group_body,
        grid_spec=grid_spec,
        out_shape=jax.ShapeDtypeStruct((CAP, H), jnp.float32),
    )(blk_e, gx, gate_w, up_w, down_w)


_CCHUNK = 16
_CNBUF = 2
_CN = TOK_PER_W // _CCHUNK


def _combine_body(y_hbm, ysh_hbm, dest_hbm, out_hbm,
                  a_v, b_v, c_v, i1_v, i2_v, gsem, osem):
    wid = lax.axis_index("s") * 2 + lax.axis_index("c")
    base = wid * TOK_PER_W

    def _issue(c, buf):
        t0 = base + c * _CCHUNK
        pltpu.sync_copy(dest_hbm.at[0, pl.ds(t0, _CCHUNK)], i1_v.at[buf])
        pltpu.sync_copy(dest_hbm.at[1, pl.ds(t0, _CCHUNK)], i2_v.at[buf])
        return (pltpu.async_copy(y_hbm.at[i1_v.at[buf]], a_v.at[buf], gsem),
                pltpu.async_copy(y_hbm.at[i2_v.at[buf]], b_v.at[buf], gsem),
                pltpu.async_copy(ysh_hbm.at[pl.ds(t0, _CCHUNK), :],
                                 c_v.at[buf], gsem))

    pend = [None] * _CN
    outcp = [None] * _CN
    for c in range(_CNBUF - 1):
        pend[c] = _issue(c, c % _CNBUF)
    for c in range(_CN):
        if c + _CNBUF - 1 < _CN:
            if c - 1 >= 0:
                outcp[c - 1].wait()
            pend[c + _CNBUF - 1] = _issue(c + _CNBUF - 1,
                                          (c + _CNBUF - 1) % _CNBUF)
        for cp in pend[c]:
            cp.wait()
        buf = c % _CNBUF

        def _row(j, _):
            for cb in range(H // 16):
                sl = pl.ds(cb * 16, 16)
                a_v[buf, j, sl] = a_v[buf, j, sl] + b_v[buf, j, sl] + c_v[buf, j, sl]
            return 0

        lax.fori_loop(0, _CCHUNK, _row, 0)
        outcp[c] = pltpu.async_copy(
            a_v.at[buf], out_hbm.at[pl.ds(base + c * _CCHUNK, _CCHUNK), :], osem)
    for c in range(max(0, _CN - _CNBUF), _CN):
        outcp[c].wait()


def _combine(y, ysh, dest):
    mesh = plsc.VectorSubcoreMesh(core_axis_name="c", subcore_axis_name="s")
    return pl.kernel(
        _combine_body,
        out_type=jax.ShapeDtypeStruct((T, H), jnp.float32),
        mesh=mesh,
        scratch_types=[
            pltpu.VMEM((_CNBUF, _CCHUNK, H), jnp.float32),
            pltpu.VMEM((_CNBUF, _CCHUNK, H), jnp.float32),
            pltpu.VMEM((_CNBUF, _CCHUNK, H), jnp.float32),
            pltpu.VMEM((_CNBUF, _CCHUNK), jnp.int32),
            pltpu.VMEM((_CNBUF, _CCHUNK), jnp.int32),
            pltpu.SemaphoreType.DMA,
            pltpu.SemaphoreType.DMA,
        ],
    )(y, ysh, dest)


def kernel(hidden_states, router_w, gate_w, up_w, down_w,
           shared_gate_w, shared_up_w, shared_down_w):
    b, s, hd = hidden_states.shape
    hs = hidden_states.reshape(-1, hd)
    router_scores, dest, sx, blk_e = _router(hs, router_w)
    ysh = _shared_mlp(hs, shared_gate_w, shared_up_w, shared_down_w)
    gx = _dispatch(sx, dest)
    y = _grouped_mlp(blk_e, gx, gate_w, up_w, down_w)
    out = _combine(y, ysh, dest)
    return out, router_scores
